# baseline (speedup 1.0000x reference)
import functools

import jax
import jax.numpy as jnp
from jax import lax
from jax.experimental import pallas as pl
from jax.experimental.pallas import tpu as pltpu
from jax.experimental.pallas import tpu_sc as plsc

B, L, D = 1024, 50, 768
N = B * L
EPS = 1e-12

NC, NS = 2, 16
NW = NC * NS
SPW = B // NW
LP = 56
NBUF = 3


def _sc_gather(ids3, table):
    mesh = plsc.VectorSubcoreMesh(core_axis_name="c", subcore_axis_name="s")

    @functools.partial(
        pl.kernel,
        mesh=mesh,
        out_type=jax.ShapeDtypeStruct((B, LP, D), jnp.float32),
        scratch_types=[
            pltpu.VMEM((SPW * LP,), jnp.int32),
            pltpu.VMEM((LP, D), jnp.float32),
            pltpu.VMEM((LP, D), jnp.float32),
            pltpu.SemaphoreType.DMA,
            pltpu.SemaphoreType.DMA,
            pltpu.SemaphoreType.DMA,
            pltpu.SemaphoreType.DMA,
        ],
    )
    def k(ids_hbm, table_hbm, out_hbm, idx_v, rows_a, rows_b, gs_a, gs_b,
          os_a, os_b):
        wid = lax.axis_index("s") * NC + lax.axis_index("c")
        seq0 = wid * SPW

        pltpu.sync_copy(ids_hbm.at[wid], idx_v)

        rows = (rows_a, rows_b)
        gsem = (gs_a, gs_b)
        osem = (os_a, os_b)

        def gather_start(p, buf):
            pltpu.make_async_copy(
                table_hbm.at[idx_v.at[pl.ds(p * LP, LP)]],
                rows[buf], gsem[buf]).start()

        def gather_wait(buf):
            pltpu.make_async_copy(
                table_hbm.at[idx_v.at[pl.ds(0, LP)]],
                rows[buf], gsem[buf]).wait()

        def out_start(p, buf):
            pltpu.make_async_copy(
                rows[buf], out_hbm.at[seq0 + p], osem[buf]).start()

        def out_wait(buf):
            pltpu.make_async_copy(
                rows[buf], out_hbm.at[seq0], osem[buf]).wait()

        gather_start(0, 0)
        gather_start(1, 1)

        def loop_body(pp, c):
            for buf in range(2):
                p = pp * 2 + buf
                gather_wait(buf)
                out_start(p, buf)
                out_wait(buf)
                gather_start(p + 2, buf)
            return c

        lax.fori_loop(0, SPW // 2 - 1, loop_body, 0)
        for buf in range(2):
            p = SPW - 2 + buf
            gather_wait(buf)
            out_start(p, buf)
            out_wait(buf)

    return k(ids3, table)


SB = 8


def _ln_body(x_ref, padd_ref, g_ref, bta_ref, o_ref):
    e = x_ref[:, :L, :] + padd_ref[...][None, :, :]
    mu = jnp.mean(e, axis=-1, keepdims=True)
    d = e - mu
    var = jnp.mean(d * d, axis=-1, keepdims=True)
    o_ref[...] = d * lax.rsqrt(var + EPS) * g_ref[...][None, :, :] \
        + bta_ref[...][None, :, :]


def _tc_add_ln(stag, padd, gamma2, beta2):
    return pl.pallas_call(
        _ln_body,
        grid=(B // SB,),
        in_specs=[
            pl.BlockSpec((SB, LP, D), lambda i: (i, 0, 0)),
            pl.BlockSpec((L, D), lambda i: (0, 0)),
            pl.BlockSpec((1, D), lambda i: (0, 0)),
            pl.BlockSpec((1, D), lambda i: (0, 0)),
        ],
        out_specs=pl.BlockSpec((SB, L, D), lambda i: (i, 0, 0)),
        out_shape=jax.ShapeDtypeStruct((B, L, D), jnp.float32),
        compiler_params=pltpu.CompilerParams(
            dimension_semantics=("arbitrary",),
        ),
    )(stag, padd, gamma2, beta2)


def kernel(input_ids, word_emb, pos_emb, type_emb, ln_gamma, ln_beta):
    ids_p = jnp.pad(input_ids.astype(jnp.int32), ((0, 0), (0, LP - L)))
    ids3 = ids_p.reshape(NW, SPW * LP)
    stag = _sc_gather(ids3, word_emb)
    padd = pos_emb[:L] + type_emb[0][None, :]
    return _tc_add_ln(stag, padd, ln_gamma.reshape(1, D),
                      ln_beta.reshape(1, D))


# speedup vs baseline: 1.0007x; 1.0007x over previous
import functools

import jax
import jax.numpy as jnp
from jax import lax
from jax.experimental import pallas as pl
from jax.experimental.pallas import tpu as pltpu
from jax.experimental.pallas import tpu_sc as plsc

B, L, D = 1024, 50, 768
N = B * L
EPS = 1e-12

NC, NS = 2, 16
NW = NC * NS
SPW = B // NW
LP = 56
NBUF = 3


def _sc_gather(ids3, table):
    mesh = plsc.VectorSubcoreMesh(core_axis_name="c", subcore_axis_name="s")

    @functools.partial(
        pl.kernel,
        mesh=mesh,
        out_type=jax.ShapeDtypeStruct((B * LP, D), jnp.float32),
        scratch_types=[
            pltpu.VMEM((SPW * LP,), jnp.int32),
            pltpu.VMEM((LP, D), jnp.float32),
            pltpu.VMEM((LP, D), jnp.float32),
            pltpu.SemaphoreType.DMA,
            pltpu.SemaphoreType.DMA,
            pltpu.SemaphoreType.DMA,
            pltpu.SemaphoreType.DMA,
        ],
    )
    def k(ids_hbm, table_hbm, out_hbm, idx_v, rows_a, rows_b, gs_a, gs_b,
          os_a, os_b):
        wid = lax.axis_index("s") * NC + lax.axis_index("c")
        seq0 = wid * SPW

        pltpu.sync_copy(ids_hbm.at[wid], idx_v)

        rows = (rows_a, rows_b)
        gsem = (gs_a, gs_b)
        osem = (os_a, os_b)

        def gather_start(p, buf):
            pltpu.make_async_copy(
                table_hbm.at[idx_v.at[pl.ds(p * LP, LP)]],
                rows[buf], gsem[buf]).start()

        def gather_wait(buf):
            pltpu.make_async_copy(
                table_hbm.at[idx_v.at[pl.ds(0, LP)]],
                rows[buf], gsem[buf]).wait()

        def out_start(p, buf):
            pltpu.make_async_copy(
                rows[buf], out_hbm.at[pl.ds((seq0 + p) * LP, LP)],
                osem[buf]).start()

        def out_wait(buf):
            pltpu.make_async_copy(
                rows[buf], out_hbm.at[pl.ds(0, LP)], osem[buf]).wait()

        gather_start(0, 0)
        gather_start(1, 1)

        def loop_body(pp, c):
            for buf in range(2):
                p = pp * 2 + buf
                gather_wait(buf)
                out_start(p, buf)
                out_wait(buf)
                gather_start(p + 2, buf)
            return c

        lax.fori_loop(0, SPW // 2 - 1, loop_body, 0)
        for buf in range(2):
            p = SPW - 2 + buf
            gather_wait(buf)
            out_start(p, buf)
            out_wait(buf)

    return k(ids3, table)


SB = 8


def _ln_body(x_ref, padd_ref, g_ref, bta_ref, o_ref):
    e = x_ref[:, :L, :] + padd_ref[...][None, :, :]
    mu = jnp.mean(e, axis=-1, keepdims=True)
    d = e - mu
    var = jnp.mean(d * d, axis=-1, keepdims=True)
    o_ref[...] = d * lax.rsqrt(var + EPS) * g_ref[...][None, :, :] \
        + bta_ref[...][None, :, :]


def _tc_add_ln(stag, padd, gamma2, beta2):
    return pl.pallas_call(
        _ln_body,
        grid=(B // SB,),
        in_specs=[
            pl.BlockSpec((SB, LP, D), lambda i: (i, 0, 0)),
            pl.BlockSpec((L, D), lambda i: (0, 0)),
            pl.BlockSpec((1, D), lambda i: (0, 0)),
            pl.BlockSpec((1, D), lambda i: (0, 0)),
        ],
        out_specs=pl.BlockSpec((SB, L, D), lambda i: (i, 0, 0)),
        out_shape=jax.ShapeDtypeStruct((B, L, D), jnp.float32),
        compiler_params=pltpu.CompilerParams(
            dimension_semantics=("arbitrary",),
        ),
    )(stag, padd, gamma2, beta2)


def kernel(input_ids, word_emb, pos_emb, type_emb, ln_gamma, ln_beta):
    ids_p = jnp.pad(input_ids.astype(jnp.int32), ((0, 0), (0, LP - L)))
    ids3 = ids_p.reshape(NW, SPW * LP)
    stag = _sc_gather(ids3, word_emb).reshape(B, LP, D)
    padd = pos_emb[:L] + type_emb[0][None, :]
    return _tc_add_ln(stag, padd, ln_gamma.reshape(1, D),
                      ln_beta.reshape(1, D))


# speedup vs baseline: 1.0042x; 1.0035x over previous
import functools

import jax
import jax.numpy as jnp
from jax import lax
from jax.experimental import pallas as pl
from jax.experimental.pallas import tpu as pltpu
from jax.experimental.pallas import tpu_sc as plsc

B, L, D = 1024, 50, 768
N = B * L
EPS = 1e-12

NC, NS = 2, 16
NW = NC * NS
SPW = B // NW
LP = 56
NBUF = 3


def _sc_gather(ids3, table):
    mesh = plsc.VectorSubcoreMesh(core_axis_name="c", subcore_axis_name="s")

    @functools.partial(
        pl.kernel,
        mesh=mesh,
        out_type=jax.ShapeDtypeStruct((B * LP, D), jnp.float32),
        scratch_types=[
            pltpu.VMEM((SPW, LP), jnp.int32),
            pltpu.VMEM((LP, D), jnp.float32),
            pltpu.VMEM((LP, D), jnp.float32),
            pltpu.SemaphoreType.DMA,
            pltpu.SemaphoreType.DMA,
            pltpu.SemaphoreType.DMA,
            pltpu.SemaphoreType.DMA,
        ],
    )
    def k(ids_hbm, table_hbm, out_hbm, idx_v, rows_a, rows_b, gs_a, gs_b,
          os_a, os_b):
        wid = lax.axis_index("s") * NC + lax.axis_index("c")
        seq0 = wid * SPW

        pltpu.sync_copy(ids_hbm.at[wid], idx_v)

        rows = (rows_a, rows_b)
        gsem = (gs_a, gs_b)
        osem = (os_a, os_b)

        def gather_start(p, buf):
            pltpu.make_async_copy(
                table_hbm.at[idx_v.at[p]],
                rows[buf], gsem[buf]).start()

        def gather_wait(buf):
            pltpu.make_async_copy(
                table_hbm.at[idx_v.at[0]],
                rows[buf], gsem[buf]).wait()

        def out_start(p, buf):
            pltpu.make_async_copy(
                rows[buf], out_hbm.at[pl.ds((seq0 + p) * LP, LP)],
                osem[buf]).start()

        def out_wait(buf):
            pltpu.make_async_copy(
                rows[buf], out_hbm.at[pl.ds(0, LP)], osem[buf]).wait()

        gather_start(0, 0)
        gather_start(1, 1)

        def loop_body(pp, c):
            for buf in range(2):
                p = pp * 2 + buf
                gather_wait(buf)
                out_start(p, buf)
                out_wait(buf)
                gather_start(p + 2, buf)
            return c

        lax.fori_loop(0, SPW // 2 - 1, loop_body, 0)
        for buf in range(2):
            p = SPW - 2 + buf
            gather_wait(buf)
            out_start(p, buf)
            out_wait(buf)

    return k(ids3, table)


SB = 8


def _ln_body(x_ref, padd_ref, g_ref, bta_ref, o_ref):
    e = x_ref[:, :L, :] + padd_ref[...][None, :, :]
    mu = jnp.mean(e, axis=-1, keepdims=True)
    d = e - mu
    var = jnp.mean(d * d, axis=-1, keepdims=True)
    o_ref[...] = d * lax.rsqrt(var + EPS) * g_ref[...][None, :, :] \
        + bta_ref[...][None, :, :]


def _tc_add_ln(stag, padd, gamma2, beta2):
    return pl.pallas_call(
        _ln_body,
        grid=(B // SB,),
        in_specs=[
            pl.BlockSpec((SB, LP, D), lambda i: (i, 0, 0)),
            pl.BlockSpec((L, D), lambda i: (0, 0)),
            pl.BlockSpec((1, D), lambda i: (0, 0)),
            pl.BlockSpec((1, D), lambda i: (0, 0)),
        ],
        out_specs=pl.BlockSpec((SB, L, D), lambda i: (i, 0, 0)),
        out_shape=jax.ShapeDtypeStruct((B, L, D), jnp.float32),
        compiler_params=pltpu.CompilerParams(
            dimension_semantics=("arbitrary",),
        ),
    )(stag, padd, gamma2, beta2)


def kernel(input_ids, word_emb, pos_emb, type_emb, ln_gamma, ln_beta):
    ids_p = jnp.pad(input_ids.astype(jnp.int32), ((0, 0), (0, LP - L)))
    ids3 = ids_p.reshape(NW, SPW, LP)
    stag = _sc_gather(ids3, word_emb).reshape(B, LP, D)
    padd = pos_emb[:L] + type_emb[0][None, :]
    return _tc_add_ln(stag, padd, ln_gamma.reshape(1, D),
                      ln_beta.reshape(1, D))


# speedup vs baseline: 1.9917x; 1.9835x over previous
import functools

import jax
import jax.numpy as jnp
from jax import lax
from jax.experimental import pallas as pl
from jax.experimental.pallas import tpu as pltpu
from jax.experimental.pallas import tpu_sc as plsc

B, L, D = 1024, 50, 768
N = B * L
EPS = 1e-12

NC, NS = 2, 16
NW = NC * NS
SPW = B // NW
LP = 56
NBUF = 3


def _sc_gather(ids3, table):
    mesh = plsc.VectorSubcoreMesh(core_axis_name="c", subcore_axis_name="s")

    @functools.partial(
        pl.kernel,
        mesh=mesh,
        out_type=jax.ShapeDtypeStruct((B * LP, D), jnp.float32),
        scratch_types=[
            pltpu.VMEM((SPW, LP), jnp.int32),
            pltpu.VMEM((LP, D), jnp.float32),
            pltpu.VMEM((LP, D), jnp.float32),
            pltpu.SemaphoreType.DMA,
            pltpu.SemaphoreType.DMA,
            pltpu.SemaphoreType.DMA,
            pltpu.SemaphoreType.DMA,
        ],
    )
    def k(ids_hbm, table_hbm, out_hbm, idx_v, rows_a, rows_b, gs_a, gs_b,
          os_a, os_b):
        wid = lax.axis_index("s") * NC + lax.axis_index("c")
        seq0 = wid * SPW

        pltpu.sync_copy(ids_hbm.at[wid], idx_v)

        rows = (rows_a, rows_b)
        gsem = (gs_a, gs_b)
        osem = (os_a, os_b)

        def gather_start(p, buf):
            pltpu.make_async_copy(
                table_hbm.at[idx_v.at[p]],
                rows[buf], gsem[buf]).start()

        def gather_wait(buf):
            pltpu.make_async_copy(
                table_hbm.at[idx_v.at[0]],
                rows[buf], gsem[buf]).wait()

        def out_start(p, buf):
            pltpu.make_async_copy(
                rows[buf], out_hbm.at[pl.ds((seq0 + p) * LP, LP)],
                osem[buf]).start()

        def out_wait(buf):
            pltpu.make_async_copy(
                rows[buf], out_hbm.at[pl.ds(0, LP)], osem[buf]).wait()

        gather_start(0, 0)
        gather_start(1, 1)

        def loop_body(pp, c):
            for buf in range(2):
                p = pp * 2 + buf
                gather_wait(buf)
                out_start(p, buf)
                out_wait(buf)
                gather_start(p + 2, buf)
            return c

        lax.fori_loop(0, SPW // 2 - 1, loop_body, 0)
        for buf in range(2):
            p = SPW - 2 + buf
            gather_wait(buf)
            out_start(p, buf)
            out_wait(buf)

    return k(ids3, table)


SB = 8


def _ln_body(x_ref, padd_ref, g_ref, bta_ref, o_ref):
    e = x_ref[:, :L, :] + padd_ref[...][None, :, :]
    mu = jnp.mean(e, axis=-1, keepdims=True)
    d = e - mu
    var = jnp.mean(d * d, axis=-1, keepdims=True)
    o_ref[...] = d * lax.rsqrt(var + EPS) * g_ref[...][None, :, :] \
        + bta_ref[...][None, :, :]


def _tc_add_ln(stag, padd, gamma2, beta2):
    return pl.pallas_call(
        _ln_body,
        grid=(B // SB,),
        in_specs=[
            pl.BlockSpec((SB, LP, D), lambda i: (i, 0, 0)),
            pl.BlockSpec((L, D), lambda i: (0, 0)),
            pl.BlockSpec((1, D), lambda i: (0, 0)),
            pl.BlockSpec((1, D), lambda i: (0, 0)),
        ],
        out_specs=pl.BlockSpec((SB, L, D), lambda i: (i, 0, 0)),
        out_shape=jax.ShapeDtypeStruct((B, L, D), jnp.float32),
        compiler_params=pltpu.CompilerParams(
            dimension_semantics=("arbitrary",),
        ),
    )(stag, padd, gamma2, beta2)


def kernel(input_ids, word_emb, pos_emb, type_emb, ln_gamma, ln_beta):
    ids32 = input_ids.astype(jnp.int32)
    ids_p = jnp.concatenate([ids32, ids32[:, :LP - L]], axis=1)
    ids3 = ids_p.reshape(NW, SPW, LP)
    stag = _sc_gather(ids3, word_emb).reshape(B, LP, D)
    padd = pos_emb[:L] + type_emb[0][None, :]
    return _tc_add_ln(stag, padd, ln_gamma.reshape(1, D),
                      ln_beta.reshape(1, D))
